# Initial kernel scaffold; baseline (speedup 1.0000x reference)
import jax
import jax.numpy as jnp
from jax import lax
from jax.experimental import pallas as pl
from jax.experimental.pallas import tpu as pltpu
from jax.experimental.pallas import tpu_sc as plsc

_NC = 2
_NS = 16
_NW = _NC * _NS
_CHUNK = 80


def _sc_agg_kernel(N, E, D, with_deg):
    EW = E // _NW
    n_chunks = EW // _CHUNK
    RW = N // _NS
    assert EW * _NW == E and n_chunks * _CHUNK == EW and RW * _NS == N

    mesh = plsc.VectorSubcoreMesh(core_axis_name="c", subcore_axis_name="s")

    out_type = [jax.ShapeDtypeStruct((_NC, N, D), jnp.float32)]
    if with_deg:
        out_type.append(jax.ShapeDtypeStruct((_NC, N, 16), jnp.float32))

    scratch = [
        pltpu.VMEM((_CHUNK,), jnp.int32),
        pltpu.VMEM((_CHUNK,), jnp.int32),
        pltpu.VMEM((_CHUNK, D), jnp.float32),
        pltpu.VMEM_SHARED((N, D), jnp.float32),
        pltpu.SemaphoreType.DMA,
    ]
    if with_deg:
        scratch.append(pltpu.VMEM((_CHUNK, 16), jnp.float32))
        scratch.append(pltpu.VMEM_SHARED((N, 16), jnp.float32))

    def body(*refs):
        if with_deg:
            (table, srci, dsti, zfeat, zdeg, ones_h,
             out_acc, out_deg,
             src_v, dst_v, rows_v, acc_sh, sem, ones_v, deg_sh) = refs
        else:
            (table, srci, dsti, zfeat,
             out_acc,
             src_v, dst_v, rows_v, acc_sh, sem) = refs
        c = lax.axis_index("c")
        s = lax.axis_index("s")
        w = c * _NS + s

        pltpu.sync_copy(zfeat, acc_sh.at[pl.ds(s * RW, RW)])
        if with_deg:
            pltpu.sync_copy(zdeg, deg_sh.at[pl.ds(s * RW, RW)])
            pltpu.sync_copy(ones_h, ones_v)
        plsc.subcore_barrier()

        base_e = w * EW

        def chunk(i, carry):
            off = base_e + i * _CHUNK
            pltpu.sync_copy(srci.at[pl.ds(off, _CHUNK)], src_v)
            pltpu.sync_copy(dsti.at[pl.ds(off, _CHUNK)], dst_v)
            pltpu.async_copy(table.at[src_v], rows_v, sem).wait()
            pltpu.sync_copy(rows_v, acc_sh.at[dst_v], add=True)
            if with_deg:
                pltpu.sync_copy(ones_v, deg_sh.at[dst_v], add=True)
            return carry

        lax.fori_loop(0, n_chunks, chunk, 0)
        plsc.subcore_barrier()

        pltpu.sync_copy(acc_sh.at[pl.ds(s * RW, RW)],
                        out_acc.at[c, pl.ds(s * RW, RW)])
        if with_deg:
            pltpu.sync_copy(deg_sh.at[pl.ds(s * RW, RW)],
                            out_deg.at[c, pl.ds(s * RW, RW)])

    return pl.kernel(body, out_type=out_type, mesh=mesh, scratch_types=scratch)


def _tc_layer(N, D, DO, relu, with_deg):
    R = 2000
    grid = (N // R,)

    def body(*refs):
        if with_deg:
            (x_r, a0_r, a1_r, d0_r, d1_r, ws_r, wn_r, b_r, o_r, inv_r) = refs
            deg = d0_r[:, 0:1] + d1_r[:, 0:1]
            inv = 1.0 / jnp.maximum(deg, 1.0)
            inv_r[...] = inv
        else:
            (x_r, a0_r, a1_r, inv_in_r, ws_r, wn_r, b_r, o_r) = refs
            inv = inv_in_r[...]
        hn = (a0_r[...] + a1_r[...]) * inv
        acc = jnp.dot(x_r[...], ws_r[...], preferred_element_type=jnp.float32)
        acc = acc + jnp.dot(hn, wn_r[...], preferred_element_type=jnp.float32)
        acc = acc + b_r[...]
        if relu:
            acc = jnp.maximum(acc, 0.0)
        o_r[...] = acc

    def row_blk(d):
        return pl.BlockSpec((R, d), lambda i: (i, 0))

    def full(shape):
        return pl.BlockSpec(shape, lambda i: (0,) * len(shape))

    if with_deg:
        in_specs = [row_blk(D), row_blk(D), row_blk(D), row_blk(16), row_blk(16),
                    full((D, DO)), full((D, DO)), full((1, DO))]
        out_shape = [jax.ShapeDtypeStruct((N, DO), jnp.float32),
                     jax.ShapeDtypeStruct((N, 1), jnp.float32)]
        out_specs = [row_blk(DO), row_blk(1)]
    else:
        in_specs = [row_blk(D), row_blk(D), row_blk(D), row_blk(1),
                    full((D, DO)), full((D, DO)), full((1, DO))]
        out_shape = jax.ShapeDtypeStruct((N, DO), jnp.float32)
        out_specs = row_blk(DO)

    return pl.pallas_call(
        body, grid=grid, in_specs=in_specs, out_specs=out_specs,
        out_shape=out_shape)


def kernel(x, edge_index, W_self1, W_neigh1, b1, W_self2, W_neigh2, b2):
    N, D = x.shape
    E = edge_index.shape[1]
    n_cls = W_self2.shape[1]

    src = edge_index[0].astype(jnp.int32)
    dst = edge_index[1].astype(jnp.int32)

    RW = N // _NS
    zfeat = jnp.zeros((RW, D), jnp.float32)
    zdeg = jnp.zeros((RW, 16), jnp.float32)
    ones_h = jnp.ones((_CHUNK, 16), jnp.float32)

    sc1 = _sc_agg_kernel(N, E, D, with_deg=True)
    sc2 = _sc_agg_kernel(N, E, D, with_deg=False)
    tc1 = _tc_layer(N, D, D, relu=True, with_deg=True)
    tc2 = _tc_layer(N, D, D, relu=False, with_deg=False)

    acc1, degp = sc1(x, src, dst, zfeat, zdeg, ones_h)
    h, inv = tc1(x, acc1[0], acc1[1], degp[0], degp[1],
                 W_self1, W_neigh1, b1.reshape(1, D))

    acc2 = sc2(h, src, dst, zfeat)
    if isinstance(acc2, (list, tuple)):
        acc2 = acc2[0]
    W_self2p = jnp.pad(W_self2, ((0, 0), (0, D - n_cls)))
    W_neigh2p = jnp.pad(W_neigh2, ((0, 0), (0, D - n_cls)))
    b2p = jnp.pad(b2, (0, D - n_cls)).reshape(1, D)
    outp = tc2(h, acc2[0], acc2[1], inv, W_self2p, W_neigh2p, b2p)
    return outp[:, :n_cls]


# speedup vs baseline: 4.2599x; 4.2599x over previous
import jax
import jax.numpy as jnp
from jax import lax
from jax.experimental import pallas as pl
from jax.experimental.pallas import tpu as pltpu
from jax.experimental.pallas import tpu_sc as plsc

_NC = 2
_NS = 16
_NW = _NC * _NS
_CHUNK = 80


def _pad_rows(N):
    return -(-N // (_NS * 8)) * (_NS * 8)


def _sc_agg_kernel(N, E, D, with_deg):
    EW = E // _NW
    n_chunks = EW // _CHUNK
    NP = _pad_rows(N)
    RW = NP // _NS
    assert EW * _NW == E and n_chunks * _CHUNK == EW

    mesh = plsc.VectorSubcoreMesh(core_axis_name="c", subcore_axis_name="s")

    out_type = [jax.ShapeDtypeStruct((_NC, NP, D), jnp.float32)]
    if with_deg:
        out_type.append(jax.ShapeDtypeStruct((_NC, NP, 128), jnp.float32))

    scratch = [
        pltpu.VMEM((_CHUNK,), jnp.int32),
        pltpu.VMEM((_CHUNK,), jnp.int32),
        pltpu.VMEM((_CHUNK, D), jnp.float32),
        pltpu.VMEM_SHARED((NP, D), jnp.float32),
        pltpu.SemaphoreType.DMA,
    ]
    if with_deg:
        scratch.append(pltpu.VMEM((_CHUNK, 128), jnp.float32))

    def body(*refs):
        if with_deg:
            (table, srci, dsti, zfeat, ones_h,
             out_acc, out_deg,
             src_v, dst_v, rows_v, acc_sh, sem, ones_v) = refs
        else:
            (table, srci, dsti, zfeat,
             out_acc,
             src_v, dst_v, rows_v, acc_sh, sem) = refs
        c = lax.axis_index("c")
        s = lax.axis_index("s")
        w = c * _NS + s
        base_e = w * EW

        def zero_acc():
            pltpu.sync_copy(zfeat, acc_sh.at[pl.ds(s * RW, RW)])

        if with_deg:
            zero_acc()
            pltpu.sync_copy(ones_h, ones_v)
            plsc.subcore_barrier()

            def deg_chunk(i, carry):
                pltpu.sync_copy(dsti.at[pl.ds(base_e + i * _CHUNK, _CHUNK)],
                                dst_v)
                pltpu.sync_copy(ones_v, acc_sh.at[dst_v], add=True)
                return carry

            lax.fori_loop(0, n_chunks, deg_chunk, 0)
            plsc.subcore_barrier()
            pltpu.sync_copy(acc_sh.at[pl.ds(s * RW, RW)],
                            out_deg.at[c, pl.ds(s * RW, RW)])
            plsc.subcore_barrier()

        zero_acc()
        plsc.subcore_barrier()

        def chunk(i, carry):
            off = base_e + i * _CHUNK
            pltpu.sync_copy(srci.at[pl.ds(off, _CHUNK)], src_v)
            pltpu.sync_copy(dsti.at[pl.ds(off, _CHUNK)], dst_v)
            pltpu.async_copy(table.at[src_v], rows_v, sem).wait()
            pltpu.sync_copy(rows_v, acc_sh.at[dst_v], add=True)
            return carry

        lax.fori_loop(0, n_chunks, chunk, 0)
        plsc.subcore_barrier()

        pltpu.sync_copy(acc_sh.at[pl.ds(s * RW, RW)],
                        out_acc.at[c, pl.ds(s * RW, RW)])

    return pl.kernel(body, out_type=out_type, mesh=mesh, scratch_types=scratch)


def _tc_layer(N, D, DO, relu, with_deg):
    R = 2000
    grid = (N // R,)

    def body(*refs):
        if with_deg:
            (x_r, a0_r, a1_r, d0_r, d1_r, ws_r, wn_r, b_r, o_r, inv_r) = refs
            deg = d0_r[:, 0:1] + d1_r[:, 0:1]
            inv = 1.0 / jnp.maximum(deg, 1.0)
            inv_r[...] = inv
        else:
            (x_r, a0_r, a1_r, inv_in_r, ws_r, wn_r, b_r, o_r) = refs
            inv = inv_in_r[...]
        hn = (a0_r[...] + a1_r[...]) * inv
        acc = jnp.dot(x_r[...], ws_r[...], preferred_element_type=jnp.float32)
        acc = acc + jnp.dot(hn, wn_r[...], preferred_element_type=jnp.float32)
        acc = acc + b_r[...]
        if relu:
            acc = jnp.maximum(acc, 0.0)
        o_r[...] = acc

    def row_blk(d):
        return pl.BlockSpec((R, d), lambda i: (i, 0))

    def full(shape):
        return pl.BlockSpec(shape, lambda i: (0,) * len(shape))

    if with_deg:
        in_specs = [row_blk(D), row_blk(D), row_blk(D), row_blk(128),
                    row_blk(128),
                    full((D, DO)), full((D, DO)), full((1, DO))]
        out_shape = [jax.ShapeDtypeStruct((N, DO), jnp.float32),
                     jax.ShapeDtypeStruct((N, 1), jnp.float32)]
        out_specs = [row_blk(DO), row_blk(1)]
    else:
        in_specs = [row_blk(D), row_blk(D), row_blk(D), row_blk(1),
                    full((D, DO)), full((D, DO)), full((1, DO))]
        out_shape = jax.ShapeDtypeStruct((N, DO), jnp.float32)
        out_specs = row_blk(DO)

    return pl.pallas_call(
        body, grid=grid, in_specs=in_specs, out_specs=out_specs,
        out_shape=out_shape)


def kernel(x, edge_index, W_self1, W_neigh1, b1, W_self2, W_neigh2, b2):
    N, D = x.shape
    E = edge_index.shape[1]
    n_cls = W_self2.shape[1]

    src = edge_index[0].astype(jnp.int32)
    dst = edge_index[1].astype(jnp.int32)

    RW = _pad_rows(N) // _NS
    zfeat = jnp.zeros((RW, D), jnp.float32)
    ones_h = jnp.ones((_CHUNK, 128), jnp.float32)

    sc1 = _sc_agg_kernel(N, E, D, with_deg=True)
    sc2 = _sc_agg_kernel(N, E, D, with_deg=False)
    tc1 = _tc_layer(N, D, D, relu=True, with_deg=True)
    tc2 = _tc_layer(N, D, D, relu=False, with_deg=False)

    acc1, degp = sc1(x, src, dst, zfeat, ones_h)
    h, inv = tc1(x, acc1[0], acc1[1], degp[0], degp[1],
                 W_self1, W_neigh1, b1.reshape(1, D))

    acc2 = sc2(h, src, dst, zfeat)
    if isinstance(acc2, (list, tuple)):
        acc2 = acc2[0]
    W_self2p = jnp.pad(W_self2, ((0, 0), (0, D - n_cls)))
    W_neigh2p = jnp.pad(W_neigh2, ((0, 0), (0, D - n_cls)))
    b2p = jnp.pad(b2, (0, D - n_cls)).reshape(1, D)
    outp = tc2(h, acc2[0], acc2[1], inv, W_self2p, W_neigh2p, b2p)
    return outp[:, :n_cls]
